# baseline (speedup 1.0000x reference)
import functools

import jax
import jax.numpy as jnp
from jax import lax
from jax.experimental import pallas as pl
from jax.experimental.pallas import tpu as pltpu
from jax.experimental.pallas import tpu_sc as plsc

NU = 50000
NI = 50000
D = 64
H = 32
PN = 50176
TRASH = 50000
NS = 16
NC = 2
RPT = PN // NS
CB = 16
RD = 4
KG = 2
NL = 2
BLK = 512


def _prep(src, dst):
    e = src.shape[0]
    order = jnp.argsort(src)
    src = src[order]
    dst = dst[order]
    per = NS * 128
    c = -(-e // per)
    c = -(-c // (2 * CB)) * (2 * CB)
    ep = NS * c * 128
    srcp = jnp.concatenate(
        [src, jnp.zeros((ep - e,), jnp.int32)]).reshape(NS, c, 128)
    dstp = jnp.concatenate(
        [dst, jnp.full((ep - e,), TRASH, jnp.int32)]).reshape(NS, c, 128)
    gsrc = jnp.stack([srcp, srcp + PN], axis=0)
    return gsrc, dstp, c


def _sc_prop(C):
    mesh = plsc.VectorSubcoreMesh(core_axis_name="c", subcore_axis_name="s")

    nb = C // CB

    @functools.partial(
        pl.kernel,
        out_type=jax.ShapeDtypeStruct((NC, PN, H), jnp.float32),
        mesh=mesh,
        scratch_types=[
            pltpu.VMEM_SHARED((PN, H), jnp.float32),
            pltpu.VMEM((CB, 128), jnp.int32),
            pltpu.VMEM((CB, 128), jnp.int32),
            pltpu.VMEM((CB, 128), jnp.int32),
            pltpu.VMEM((CB, 128), jnp.int32),
            pltpu.VMEM((RD, 128, H), jnp.float32),
            pltpu.SemaphoreType.DMA((RD,)),
            pltpu.SemaphoreType.DMA((RD,)),
            pltpu.SemaphoreType.DMA((4,)),
        ],
        compiler_params=pltpu.CompilerParams(use_tc_tiling_on_sc=False),
    )
    def k(table, gsrc, gdst, zeros, out, acc, sba, sbb, dba, dbb, rb,
          gsem, ssem, isem):
        c = lax.axis_index("c")
        s = lax.axis_index("s")
        r0 = s * RPT
        pltpu.sync_copy(zeros.at[pl.ds(r0, RPT)], acc.at[pl.ds(r0, RPT)])
        plsc.subcore_barrier()

        def stage(jb, sb, db, i0):
            pltpu.async_copy(gsrc.at[c, s, pl.ds(jb * CB, CB)], sb,
                             isem.at[i0])
            pltpu.async_copy(gdst.at[s, pl.ds(jb * CB, CB)], db,
                             isem.at[i0 + 1])

        def wait_stage(jb, sb, db, i0):
            pltpu.make_async_copy(gsrc.at[c, s, pl.ds(jb * CB, CB)], sb,
                                  isem.at[i0]).wait()
            pltpu.make_async_copy(gdst.at[s, pl.ds(jb * CB, CB)], db,
                                  isem.at[i0 + 1]).wait()

        stage(0, sba, dba, 0)

        @pl.loop(0, nb // 2)
        def _(jb2):
            jb0 = 2 * jb2
            for par, sb, db, i0, nsb, ndb, ni0 in (
                    (0, sba, dba, 0, sbb, dbb, 2),
                    (1, sbb, dbb, 2, sba, dba, 0)):
                jb = jb0 + par
                wait_stage(jb, sb, db, i0)
                gds = [None] * CB
                sds = [None] * CB
                for jj in range(CB):
                    if jj >= RD:
                        sds[jj - RD].wait()
                    gds[jj] = pltpu.async_copy(
                        table.at[sb.at[jj]], rb.at[jj % RD],
                        gsem.at[jj % RD])
                    if jj == 0:
                        if par == 0:
                            stage(jb + 1, nsb, ndb, ni0)
                        else:
                            @pl.when(jb2 < nb // 2 - 1)
                            def _():
                                stage(jb + 1, nsb, ndb, ni0)
                    if jj >= KG:
                        jk = jj - KG
                        gds[jk].wait()
                        sds[jk] = pltpu.async_copy(
                            rb.at[jk % RD], acc.at[db.at[jk]],
                            ssem.at[jk % RD], add=True)
                for jk in range(CB - KG, CB):
                    gds[jk].wait()
                    sds[jk] = pltpu.async_copy(
                        rb.at[jk % RD], acc.at[db.at[jk]],
                        ssem.at[jk % RD], add=True)
                for jk in range(CB - RD, CB):
                    sds[jk].wait()

        plsc.subcore_barrier()
        pltpu.sync_copy(acc.at[pl.ds(r0, RPT)], out.at[c, pl.ds(r0, RPT)])

    return k


def _sc_deg(Cs, Cu):
    mesh = plsc.VectorSubcoreMesh(core_axis_name="c", subcore_axis_name="s")

    @functools.partial(
        pl.kernel,
        out_type=jax.ShapeDtypeStruct((4, PN, 16), jnp.float32),
        mesh=mesh,
        scratch_types=[
            pltpu.VMEM_SHARED((PN, 16), jnp.float32),
            pltpu.VMEM_SHARED((PN, 16), jnp.float32),
            pltpu.VMEM((CB, 128), jnp.int32),
            pltpu.VMEM((128, 16), jnp.float32),
        ],
        compiler_params=pltpu.CompilerParams(use_tc_tiling_on_sc=False),
    )
    def k(d_sf, d_sb, d_uf, d_ub, zeros16, ones16, out, acc0, acc1, dbuf, onev):
        c = lax.axis_index("c")
        s = lax.axis_index("s")
        r0 = s * RPT
        pltpu.sync_copy(ones16, onev)
        pltpu.sync_copy(zeros16.at[pl.ds(r0, RPT)], acc0.at[pl.ds(r0, RPT)])
        pltpu.sync_copy(zeros16.at[pl.ds(r0, RPT)], acc1.at[pl.ds(r0, RPT)])
        plsc.subcore_barrier()

        def scan(dref, accr, cc):
            @pl.loop(0, cc // CB)
            def _(jb):
                pltpu.sync_copy(dref.at[s, pl.ds(jb * CB, CB)], dbuf)
                for jj in range(CB):
                    pltpu.sync_copy(onev, accr.at[dbuf.at[jj]], add=True)

        @pl.when(c == 0)
        def _():
            scan(d_sf, acc0, Cs)
            scan(d_sb, acc1, Cs)

        @pl.when(c == 1)
        def _():
            scan(d_uf, acc0, Cu)
            scan(d_ub, acc1, Cu)

        plsc.subcore_barrier()
        pltpu.sync_copy(acc0.at[pl.ds(r0, RPT)], out.at[2 * c, pl.ds(r0, RPT)])
        pltpu.sync_copy(acc1.at[pl.ds(r0, RPT)],
                        out.at[2 * c + 1, pl.ds(r0, RPT)])

    return k


def _halves(x):
    return jnp.stack([x[:, :H], x[:, H:]], axis=0)


def _tc_gate(uep, iep, gw1, gb1, gw2, gb2, gw3, gb3, sgw, sgb):
    grid = (PN // BLK,)
    wspec = pl.BlockSpec((D, D), lambda r: (0, 0))
    bspec = pl.BlockSpec((1, D), lambda r: (0, 0))
    espec = pl.BlockSpec((BLK, D), lambda r: (r, 0))
    ospec = pl.BlockSpec((NC, BLK, H), lambda r: (0, r, 0))
    oshape = jax.ShapeDtypeStruct((NC, PN, H), jnp.float32)

    def body(ue, ie, w1, b1, w2, b2, w3, b3, sw, sb, o1, o2, o3, oi):
        x = ue[...]
        y = ie[...]

        def gate(v, w, b):
            z = jnp.dot(v, w[...], preferred_element_type=jnp.float32) + b[...]
            return v * jax.nn.sigmoid(z)

        o1[...] = _halves(gate(x, w1, b1))
        o2[...] = _halves(gate(x, w2, b2))
        o3[...] = _halves(gate(x, w3, b3))
        oi[...] = _halves(gate(y, sw, sb))

    return pl.pallas_call(
        body,
        grid=grid,
        in_specs=[espec, espec, wspec, bspec, wspec, bspec, wspec, bspec,
                  wspec, bspec],
        out_specs=[ospec, ospec, ospec, ospec],
        out_shape=[oshape, oshape, oshape, oshape],
    )(uep, iep, gw1, gb1.reshape(1, D), gw2, gb2.reshape(1, D),
      gw3, gb3.reshape(1, D), sgw, sgb.reshape(1, D))


def _tc_att(u1a, u2a, u3a, uia, deg, am, av, uacc):
    grid = (PN // BLK,)
    tspec = pl.BlockSpec((NC, BLK, H), lambda r: (0, r, 0))
    dspec = pl.BlockSpec((4, BLK, 16), lambda r: (0, r, 0))
    mspec = pl.BlockSpec((D, D), lambda r: (0, 0))
    vspec = pl.BlockSpec((1, D), lambda r: (0, 0))
    oshape = jax.ShapeDtypeStruct((NC, PN, H), jnp.float32)

    def body(u1r, u2r, u3r, uir, dr, amr, avr, uar,
             o1, o2, o3, om, oa):
        d1 = 1.0 / jnp.maximum(dr[0, :, 0:1], 1.0)
        d2 = 1.0 / jnp.maximum(dr[1, :, 0:1], 1.0)
        du = 1.0 / jnp.maximum(dr[3, :, 0:1], 1.0)
        u1 = jnp.concatenate([u1r[0], u1r[1]], axis=1) * d1
        u2 = jnp.concatenate([u2r[0], u2r[1]], axis=1) * d2
        u3 = jnp.concatenate([u3r[0], u3r[1]], axis=1) * d1
        a = jnp.dot(amr[...], avr[...].T,
                    preferred_element_type=jnp.float32)
        w1 = jnp.dot(u1, a, preferred_element_type=jnp.float32)
        w2 = jnp.dot(u2, a, preferred_element_type=jnp.float32)
        w3 = jnp.dot(u3, a, preferred_element_type=jnp.float32)
        m = jnp.maximum(jnp.maximum(w1, w2), w3)
        e1 = jnp.exp(w1 - m)
        e2 = jnp.exp(w2 - m)
        e3 = jnp.exp(w3 - m)
        den = e1 + e2 + e3
        mixed = (u1 * e1 + u2 * e2 + u3 * e3) / den
        uf = jnp.concatenate([uir[0], uir[1]], axis=1) * du
        hm = _halves(mixed)
        o1[...] = _halves(u1)
        o2[...] = _halves(u2)
        o3[...] = _halves(u3)
        om[...] = hm
        oa[...] = uar[...] + hm + _halves(uf)

    return pl.pallas_call(
        body,
        grid=grid,
        in_specs=[tspec, tspec, tspec, tspec, dspec, mspec, vspec, tspec],
        out_specs=[tspec, tspec, tspec, tspec, tspec],
        out_shape=[oshape, oshape, oshape, oshape, oshape],
    )(u1a, u2a, u3a, uia, deg, am, av, uacc)


def _tc_item(ia, deg, iacc):
    grid = (PN // BLK,)
    tspec = pl.BlockSpec((NC, BLK, H), lambda r: (0, r, 0))
    dspec = pl.BlockSpec((4, BLK, 16), lambda r: (0, r, 0))
    oshape = jax.ShapeDtypeStruct((NC, PN, H), jnp.float32)

    def body(iar, dr, acr, oe, oa):
        di = 1.0 / jnp.maximum(dr[2, :, 0:1], 1.0)
        oe[...] = iar[...] * di[None]
        oa[...] = acr[...] + oe[...]

    return pl.pallas_call(
        body,
        grid=grid,
        in_specs=[tspec, dspec, tspec],
        out_specs=[tspec, tspec],
        out_shape=[oshape, oshape],
    )(ia, deg, iacc)


def _tc_final(uacc, iacc):
    grid = (PN // BLK,)
    tspec = pl.BlockSpec((NC, BLK, H), lambda r: (0, r, 0))
    ospec = pl.BlockSpec((BLK, D), lambda r: (r, 0))
    oshape = jax.ShapeDtypeStruct((PN, D), jnp.float32)

    def body(ur, ir, ou, oi):
        inv = 1.0 / NL
        ou[...] = jnp.concatenate([ur[0], ur[1]], axis=1) * inv
        oi[...] = jnp.concatenate([ir[0], ir[1]], axis=1) * inv

    return pl.pallas_call(
        body,
        grid=grid,
        in_specs=[tspec, tspec],
        out_specs=[ospec, ospec],
        out_shape=[oshape, oshape],
    )(uacc, iacc)


def kernel(user_emb, item_emb, gw1, gb1, gw2, gb2, gw3, gb3, sgw, sgb,
           att_mat, att_vec, social_src, social_dst, ui_user, ui_item):
    uep = jnp.zeros((PN, D), jnp.float32).at[:NU].set(user_emb)
    iep = jnp.zeros((PN, D), jnp.float32).at[:NI].set(item_emb)

    g_sf_s, g_sf_d, cs = _prep(social_src, social_dst)
    g_sb_s, g_sb_d, _ = _prep(social_dst, social_src)
    g_uf_s, g_uf_d, cu = _prep(ui_user, ui_item)
    g_ub_s, g_ub_d, _ = _prep(ui_item, ui_user)

    zeros32 = jnp.zeros((PN, H), jnp.float32)
    zeros16 = jnp.zeros((PN, 16), jnp.float32)
    ones16 = jnp.ones((128, 16), jnp.float32)

    u1, u2, u3, ie = _tc_gate(uep, iep, gw1, gb1, gw2, gb2, gw3, gb3,
                              sgw, sgb)
    deg = _sc_deg(cs, cu)(g_sf_d, g_sb_d, g_uf_d, g_ub_d, zeros16, ones16)

    prop_s = _sc_prop(cs)
    prop_u = _sc_prop(cu)

    uacc = jnp.zeros((NC, PN, H), jnp.float32)
    iacc = jnp.zeros((NC, PN, H), jnp.float32)
    for _ in range(NL):
        u1a = prop_s(u1.reshape(NC * PN, H), g_sf_s, g_sf_d, zeros32)
        u2a = prop_s(u2.reshape(NC * PN, H), g_sb_s, g_sb_d, zeros32)
        u3a = prop_s(u3.reshape(NC * PN, H), g_sf_s, g_sf_d, zeros32)
        uia = prop_u(ie.reshape(NC * PN, H), g_ub_s, g_ub_d, zeros32)
        u1, u2, u3, mixed, uacc = _tc_att(u1a, u2a, u3a, uia, deg,
                                          att_mat, att_vec, uacc)
        ia = prop_u(mixed.reshape(NC * PN, H), g_uf_s, g_uf_d, zeros32)
        ie, iacc = _tc_item(ia, deg, iacc)

    up, ip = _tc_final(uacc, iacc)
    return jnp.concatenate([up[:NU], ip[:NI]], axis=0)


# speedup vs baseline: 1.5268x; 1.5268x over previous
import functools

import jax
import jax.numpy as jnp
from jax import lax
from jax.experimental import pallas as pl
from jax.experimental.pallas import tpu as pltpu
from jax.experimental.pallas import tpu_sc as plsc

NU = 50000
NI = 50000
D = 64
H = 32
PN = 50176
TRASH = 50000
NS = 16
NC = 2
RPT = PN // NS
CB = 8
NL = 2
BLK = 512


def _prep(src, dst):
    e = src.shape[0]
    per = NS * 128
    c = -(-e // per)
    c = -(-c // CB) * CB
    ep = NS * c * 128
    srcp = jnp.concatenate(
        [src, jnp.zeros((ep - e,), jnp.int32)]).reshape(NS, c, 128)
    dstp = jnp.concatenate(
        [dst, jnp.full((ep - e,), TRASH, jnp.int32)]).reshape(NS, c, 128)
    gsrc = jnp.stack([srcp, srcp + PN], axis=0)
    return gsrc, dstp, c


def _sc_prop_multi(cc_list):
    mesh = plsc.VectorSubcoreMesh(core_axis_name="c", subcore_axis_name="s")
    n = len(cc_list)

    @functools.partial(
        pl.kernel,
        out_type=[jax.ShapeDtypeStruct((NC, PN, H), jnp.float32)] * n,
        mesh=mesh,
        scratch_types=[
            pltpu.VMEM_SHARED((PN, H), jnp.float32),
            pltpu.VMEM((CB, 128), jnp.int32),
            pltpu.VMEM((CB, 128), jnp.int32),
            pltpu.VMEM((128, H), jnp.float32),
            pltpu.VMEM((128, H), jnp.float32),
            pltpu.SemaphoreType.DMA,
            pltpu.SemaphoreType.DMA,
        ],
        compiler_params=pltpu.CompilerParams(use_tc_tiling_on_sc=False),
    )
    def k(*refs):
        tables = refs[0:3 * n:3]
        gsrcs = refs[1:3 * n:3]
        gdsts = refs[2:3 * n:3]
        zeros = refs[3 * n]
        outs = refs[3 * n + 1:3 * n + 1 + n]
        acc, sbuf, dbuf, rba, rbb, sma, smb = refs[3 * n + 1 + n:]
        c = lax.axis_index("c")
        s = lax.axis_index("s")
        r0 = s * RPT
        for p in range(n):
            pltpu.sync_copy(zeros.at[pl.ds(r0, RPT)], acc.at[pl.ds(r0, RPT)])
            plsc.subcore_barrier()

            @pl.loop(0, cc_list[p] // CB)
            def _(jb, _t=tables[p], _gs=gsrcs[p], _gd=gdsts[p]):
                j0 = jb * CB
                pltpu.sync_copy(_gs.at[c, s, pl.ds(j0, CB)], sbuf)
                pltpu.sync_copy(_gd.at[s, pl.ds(j0, CB)], dbuf)
                for jj in range(CB):
                    rb = rba if jj % 2 == 0 else rbb
                    sm = sma if jj % 2 == 0 else smb
                    pltpu.async_copy(_t.at[sbuf.at[jj]], rb, sm).wait()
                    pltpu.sync_copy(rb, acc.at[dbuf.at[jj]], add=True)

            plsc.subcore_barrier()
            pltpu.sync_copy(acc.at[pl.ds(r0, RPT)],
                            outs[p].at[c, pl.ds(r0, RPT)])

    return k


def _sc_deg(Cs, Cu):
    mesh = plsc.VectorSubcoreMesh(core_axis_name="c", subcore_axis_name="s")

    @functools.partial(
        pl.kernel,
        out_type=jax.ShapeDtypeStruct((4, PN, 16), jnp.float32),
        mesh=mesh,
        scratch_types=[
            pltpu.VMEM_SHARED((PN, 16), jnp.float32),
            pltpu.VMEM_SHARED((PN, 16), jnp.float32),
            pltpu.VMEM((CB, 128), jnp.int32),
            pltpu.VMEM((128, 16), jnp.float32),
        ],
        compiler_params=pltpu.CompilerParams(use_tc_tiling_on_sc=False),
    )
    def k(d_sf, d_sb, d_uf, d_ub, zeros16, ones16, out, acc0, acc1, dbuf, onev):
        c = lax.axis_index("c")
        s = lax.axis_index("s")
        r0 = s * RPT
        pltpu.sync_copy(ones16, onev)
        pltpu.sync_copy(zeros16.at[pl.ds(r0, RPT)], acc0.at[pl.ds(r0, RPT)])
        pltpu.sync_copy(zeros16.at[pl.ds(r0, RPT)], acc1.at[pl.ds(r0, RPT)])
        plsc.subcore_barrier()

        def scan(dref, accr, cc):
            @pl.loop(0, cc // CB)
            def _(jb):
                pltpu.sync_copy(dref.at[s, pl.ds(jb * CB, CB)], dbuf)
                for jj in range(CB):
                    pltpu.sync_copy(onev, accr.at[dbuf.at[jj]], add=True)

        @pl.when(c == 0)
        def _():
            scan(d_sf, acc0, Cs)
            scan(d_sb, acc1, Cs)

        @pl.when(c == 1)
        def _():
            scan(d_uf, acc0, Cu)
            scan(d_ub, acc1, Cu)

        plsc.subcore_barrier()
        pltpu.sync_copy(acc0.at[pl.ds(r0, RPT)], out.at[2 * c, pl.ds(r0, RPT)])
        pltpu.sync_copy(acc1.at[pl.ds(r0, RPT)],
                        out.at[2 * c + 1, pl.ds(r0, RPT)])

    return k


def _halves(x):
    return jnp.stack([x[:, :H], x[:, H:]], axis=0)


def _tc_gate(uep, iep, gw1, gb1, gw2, gb2, gw3, gb3, sgw, sgb):
    grid = (PN // BLK,)
    wspec = pl.BlockSpec((D, D), lambda r: (0, 0))
    bspec = pl.BlockSpec((1, D), lambda r: (0, 0))
    espec = pl.BlockSpec((BLK, D), lambda r: (r, 0))
    ospec = pl.BlockSpec((NC, BLK, H), lambda r: (0, r, 0))
    oshape = jax.ShapeDtypeStruct((NC, PN, H), jnp.float32)

    def body(ue, ie, w1, b1, w2, b2, w3, b3, sw, sb, o1, o2, o3, oi):
        x = ue[...]
        y = ie[...]

        def gate(v, w, b):
            z = jnp.dot(v, w[...], preferred_element_type=jnp.float32) + b[...]
            return v * jax.nn.sigmoid(z)

        o1[...] = _halves(gate(x, w1, b1))
        o2[...] = _halves(gate(x, w2, b2))
        o3[...] = _halves(gate(x, w3, b3))
        oi[...] = _halves(gate(y, sw, sb))

    return pl.pallas_call(
        body,
        grid=grid,
        in_specs=[espec, espec, wspec, bspec, wspec, bspec, wspec, bspec,
                  wspec, bspec],
        out_specs=[ospec, ospec, ospec, ospec],
        out_shape=[oshape, oshape, oshape, oshape],
    )(uep, iep, gw1, gb1.reshape(1, D), gw2, gb2.reshape(1, D),
      gw3, gb3.reshape(1, D), sgw, sgb.reshape(1, D))


def _tc_att(u1a, u2a, u3a, uia, deg, am, av, uacc):
    grid = (PN // BLK,)
    tspec = pl.BlockSpec((NC, BLK, H), lambda r: (0, r, 0))
    dspec = pl.BlockSpec((4, BLK, 16), lambda r: (0, r, 0))
    mspec = pl.BlockSpec((D, D), lambda r: (0, 0))
    vspec = pl.BlockSpec((1, D), lambda r: (0, 0))
    oshape = jax.ShapeDtypeStruct((NC, PN, H), jnp.float32)

    def body(u1r, u2r, u3r, uir, dr, amr, avr, uar,
             o1, o2, o3, om, oa):
        d1 = 1.0 / jnp.maximum(dr[0, :, 0:1], 1.0)
        d2 = 1.0 / jnp.maximum(dr[1, :, 0:1], 1.0)
        du = 1.0 / jnp.maximum(dr[3, :, 0:1], 1.0)
        u1 = jnp.concatenate([u1r[0], u1r[1]], axis=1) * d1
        u2 = jnp.concatenate([u2r[0], u2r[1]], axis=1) * d2
        u3 = jnp.concatenate([u3r[0], u3r[1]], axis=1) * d1
        a = jnp.dot(amr[...], avr[...].T,
                    preferred_element_type=jnp.float32)
        w1 = jnp.dot(u1, a, preferred_element_type=jnp.float32)
        w2 = jnp.dot(u2, a, preferred_element_type=jnp.float32)
        w3 = jnp.dot(u3, a, preferred_element_type=jnp.float32)
        m = jnp.maximum(jnp.maximum(w1, w2), w3)
        e1 = jnp.exp(w1 - m)
        e2 = jnp.exp(w2 - m)
        e3 = jnp.exp(w3 - m)
        den = e1 + e2 + e3
        mixed = (u1 * e1 + u2 * e2 + u3 * e3) / den
        uf = jnp.concatenate([uir[0], uir[1]], axis=1) * du
        hm = _halves(mixed)
        o1[...] = _halves(u1)
        o2[...] = _halves(u2)
        o3[...] = _halves(u3)
        om[...] = hm
        oa[...] = uar[...] + hm + _halves(uf)

    return pl.pallas_call(
        body,
        grid=grid,
        in_specs=[tspec, tspec, tspec, tspec, dspec, mspec, vspec, tspec],
        out_specs=[tspec, tspec, tspec, tspec, tspec],
        out_shape=[oshape, oshape, oshape, oshape, oshape],
    )(u1a, u2a, u3a, uia, deg, am, av, uacc)


def _tc_item(ia, deg, iacc):
    grid = (PN // BLK,)
    tspec = pl.BlockSpec((NC, BLK, H), lambda r: (0, r, 0))
    dspec = pl.BlockSpec((4, BLK, 16), lambda r: (0, r, 0))
    oshape = jax.ShapeDtypeStruct((NC, PN, H), jnp.float32)

    def body(iar, dr, acr, oe, oa):
        di = 1.0 / jnp.maximum(dr[2, :, 0:1], 1.0)
        oe[...] = iar[...] * di[None]
        oa[...] = acr[...] + oe[...]

    return pl.pallas_call(
        body,
        grid=grid,
        in_specs=[tspec, dspec, tspec],
        out_specs=[tspec, tspec],
        out_shape=[oshape, oshape],
    )(ia, deg, iacc)


def _tc_final(uacc, iacc):
    grid = (PN // BLK,)
    tspec = pl.BlockSpec((NC, BLK, H), lambda r: (0, r, 0))
    ospec = pl.BlockSpec((BLK, D), lambda r: (r, 0))
    oshape = jax.ShapeDtypeStruct((PN, D), jnp.float32)

    def body(ur, ir, ou, oi):
        inv = 1.0 / NL
        ou[...] = jnp.concatenate([ur[0], ur[1]], axis=1) * inv
        oi[...] = jnp.concatenate([ir[0], ir[1]], axis=1) * inv

    return pl.pallas_call(
        body,
        grid=grid,
        in_specs=[tspec, tspec],
        out_specs=[ospec, ospec],
        out_shape=[oshape, oshape],
    )(uacc, iacc)


def kernel(user_emb, item_emb, gw1, gb1, gw2, gb2, gw3, gb3, sgw, sgb,
           att_mat, att_vec, social_src, social_dst, ui_user, ui_item):
    uep = jnp.zeros((PN, D), jnp.float32).at[:NU].set(user_emb)
    iep = jnp.zeros((PN, D), jnp.float32).at[:NI].set(item_emb)

    g_sf_s, g_sf_d, cs = _prep(social_src, social_dst)
    g_sb_s, g_sb_d, _ = _prep(social_dst, social_src)
    g_uf_s, g_uf_d, cu = _prep(ui_user, ui_item)
    g_ub_s, g_ub_d, _ = _prep(ui_item, ui_user)

    zeros32 = jnp.zeros((PN, H), jnp.float32)
    zeros16 = jnp.zeros((PN, 16), jnp.float32)
    ones16 = jnp.ones((128, 16), jnp.float32)

    u1, u2, u3, ie = _tc_gate(uep, iep, gw1, gb1, gw2, gb2, gw3, gb3,
                              sgw, sgb)
    deg = _sc_deg(cs, cu)(g_sf_d, g_sb_d, g_uf_d, g_ub_d, zeros16, ones16)

    prop_a = _sc_prop_multi([cs, cs, cs, cu])
    prop_b = _sc_prop_multi([cu])

    uacc = jnp.zeros((NC, PN, H), jnp.float32)
    iacc = jnp.zeros((NC, PN, H), jnp.float32)
    for _ in range(NL):
        u1a, u2a, u3a, uia = prop_a(
            u1.reshape(NC * PN, H), g_sf_s, g_sf_d,
            u2.reshape(NC * PN, H), g_sb_s, g_sb_d,
            u3.reshape(NC * PN, H), g_sf_s, g_sf_d,
            ie.reshape(NC * PN, H), g_ub_s, g_ub_d,
            zeros32)
        u1, u2, u3, mixed, uacc = _tc_att(u1a, u2a, u3a, uia, deg,
                                          att_mat, att_vec, uacc)
        (ia,) = prop_b(mixed.reshape(NC * PN, H), g_uf_s, g_uf_d, zeros32)
        ie, iacc = _tc_item(ia, deg, iacc)

    up, ip = _tc_final(uacc, iacc)
    return jnp.concatenate([up[:NU], ip[:NI]], axis=0)


# speedup vs baseline: 1.5691x; 1.0277x over previous
import functools

import jax
import jax.numpy as jnp
from jax import lax
from jax.experimental import pallas as pl
from jax.experimental.pallas import tpu as pltpu
from jax.experimental.pallas import tpu_sc as plsc

NU = 50000
NI = 50000
D = 64
H = 32
PN = 50176
TRASH = 50000
NS = 16
NC = 2
RPT = PN // NS
CB = 8
NL = 2
BLK = 512


def _prep(src, dst):
    e = src.shape[0]
    per = NS * 128
    c = -(-e // per)
    c = -(-c // CB) * CB
    ep = NS * c * 128
    srcp = jnp.concatenate(
        [src, jnp.zeros((ep - e,), jnp.int32)]).reshape(NS, c, 128)
    dstp = jnp.concatenate(
        [dst, jnp.full((ep - e,), TRASH, jnp.int32)]).reshape(NS, c, 128)
    gsrc = jnp.stack([srcp, srcp + PN], axis=0)
    return gsrc, dstp, c


def _sc_prop_multi(cc_list):
    mesh = plsc.VectorSubcoreMesh(core_axis_name="c", subcore_axis_name="s")
    n = len(cc_list)

    @functools.partial(
        pl.kernel,
        out_type=[jax.ShapeDtypeStruct((NC, PN, H), jnp.float32)] * n,
        mesh=mesh,
        scratch_types=[
            pltpu.VMEM_SHARED((PN, H), jnp.float32),
            pltpu.VMEM((CB, 128), jnp.int32),
            pltpu.VMEM((CB, 128), jnp.int32),
            pltpu.VMEM((128, H), jnp.float32),
            pltpu.VMEM((128, H), jnp.float32),
            pltpu.SemaphoreType.DMA,
            pltpu.SemaphoreType.DMA,
        ],
        compiler_params=pltpu.CompilerParams(use_tc_tiling_on_sc=False),
    )
    def k(*refs):
        tables = refs[0:3 * n:3]
        gsrcs = refs[1:3 * n:3]
        gdsts = refs[2:3 * n:3]
        zeros = refs[3 * n]
        outs = refs[3 * n + 1:3 * n + 1 + n]
        acc, sbuf, dbuf, rba, rbb, sma, smb = refs[3 * n + 1 + n:]
        c = lax.axis_index("c")
        s = lax.axis_index("s")
        r0 = s * RPT
        for p in range(n):
            pltpu.sync_copy(zeros.at[pl.ds(r0, RPT)], acc.at[pl.ds(r0, RPT)])
            plsc.subcore_barrier()

            @pl.loop(0, cc_list[p] // CB)
            def _(jb, _t=tables[p], _gs=gsrcs[p], _gd=gdsts[p]):
                j0 = jb * CB
                pltpu.sync_copy(_gs.at[c, s, pl.ds(j0, CB)], sbuf)
                pltpu.sync_copy(_gd.at[s, pl.ds(j0, CB)], dbuf)
                for jj in range(CB):
                    rb = rba if jj % 2 == 0 else rbb
                    sm = sma if jj % 2 == 0 else smb
                    pltpu.async_copy(_t.at[sbuf.at[jj]], rb, sm).wait()
                    pltpu.sync_copy(rb, acc.at[dbuf.at[jj]], add=True)

            plsc.subcore_barrier()
            pltpu.sync_copy(acc.at[pl.ds(r0, RPT)],
                            outs[p].at[c, pl.ds(r0, RPT)])

    return k


def _sc_deg(Cs, Cu):
    mesh = plsc.VectorSubcoreMesh(core_axis_name="c", subcore_axis_name="s")

    @functools.partial(
        pl.kernel,
        out_type=jax.ShapeDtypeStruct((4, PN, 16), jnp.float32),
        mesh=mesh,
        scratch_types=[
            pltpu.VMEM_SHARED((PN, 16), jnp.float32),
            pltpu.VMEM_SHARED((PN, 16), jnp.float32),
            pltpu.VMEM((CB, 128), jnp.int32),
            pltpu.VMEM((128, 16), jnp.float32),
        ],
        compiler_params=pltpu.CompilerParams(use_tc_tiling_on_sc=False),
    )
    def k(d_sf, d_sb, d_uf, d_ub, zeros16, ones16, out, acc0, acc1, dbuf, onev):
        c = lax.axis_index("c")
        s = lax.axis_index("s")
        r0 = s * RPT
        pltpu.sync_copy(ones16, onev)
        pltpu.sync_copy(zeros16.at[pl.ds(r0, RPT)], acc0.at[pl.ds(r0, RPT)])
        pltpu.sync_copy(zeros16.at[pl.ds(r0, RPT)], acc1.at[pl.ds(r0, RPT)])
        plsc.subcore_barrier()

        def scan(dref, accr, cc):
            @pl.loop(0, cc // CB)
            def _(jb):
                pltpu.sync_copy(dref.at[s, pl.ds(jb * CB, CB)], dbuf)
                for jj in range(CB):
                    pltpu.sync_copy(onev, accr.at[dbuf.at[jj]], add=True)

        @pl.when(c == 0)
        def _():
            scan(d_sf, acc0, Cs)
            scan(d_sb, acc1, Cs)

        @pl.when(c == 1)
        def _():
            scan(d_uf, acc0, Cu)
            scan(d_ub, acc1, Cu)

        plsc.subcore_barrier()
        pltpu.sync_copy(acc0.at[pl.ds(r0, RPT)], out.at[2 * c, pl.ds(r0, RPT)])
        pltpu.sync_copy(acc1.at[pl.ds(r0, RPT)],
                        out.at[2 * c + 1, pl.ds(r0, RPT)])

    return k


def _halves(x):
    return jnp.stack([x[:, :H], x[:, H:]], axis=0)


def _tc_gate(uep, iep, gw1, gb1, gw2, gb2, gw3, gb3, sgw, sgb):
    grid = (PN // BLK,)
    wspec = pl.BlockSpec((D, D), lambda r: (0, 0))
    bspec = pl.BlockSpec((1, D), lambda r: (0, 0))
    espec = pl.BlockSpec((BLK, D), lambda r: (r, 0))
    ospec = pl.BlockSpec((NC, BLK, H), lambda r: (0, r, 0))
    oshape = jax.ShapeDtypeStruct((NC, PN, H), jnp.float32)

    def body(ue, ie, w1, b1, w2, b2, w3, b3, sw, sb, o1, o2, o3, oi):
        x = ue[...]
        y = ie[...]

        def gate(v, w, b):
            z = jnp.dot(v, w[...], preferred_element_type=jnp.float32) + b[...]
            return v * jax.nn.sigmoid(z)

        o1[...] = _halves(gate(x, w1, b1))
        o2[...] = _halves(gate(x, w2, b2))
        o3[...] = _halves(gate(x, w3, b3))
        oi[...] = _halves(gate(y, sw, sb))

    return pl.pallas_call(
        body,
        grid=grid,
        in_specs=[espec, espec, wspec, bspec, wspec, bspec, wspec, bspec,
                  wspec, bspec],
        out_specs=[ospec, ospec, ospec, ospec],
        out_shape=[oshape, oshape, oshape, oshape],
    )(uep, iep, gw1, gb1.reshape(1, D), gw2, gb2.reshape(1, D),
      gw3, gb3.reshape(1, D), sgw, sgb.reshape(1, D))


def _tc_att(u1a, u2a, u3a, uia, deg, am, av, uacc):
    grid = (PN // BLK,)
    tspec = pl.BlockSpec((NC, BLK, H), lambda r: (0, r, 0))
    dspec = pl.BlockSpec((4, BLK, 16), lambda r: (0, r, 0))
    mspec = pl.BlockSpec((D, D), lambda r: (0, 0))
    vspec = pl.BlockSpec((1, D), lambda r: (0, 0))
    oshape = jax.ShapeDtypeStruct((NC, PN, H), jnp.float32)

    def body(u1r, u2r, u3r, uir, dr, amr, avr, uar,
             o1, o2, o3, om, oa):
        d1 = 1.0 / jnp.maximum(dr[0, :, 0:1], 1.0)
        d2 = 1.0 / jnp.maximum(dr[1, :, 0:1], 1.0)
        du = 1.0 / jnp.maximum(dr[3, :, 0:1], 1.0)
        u1 = jnp.concatenate([u1r[0], u1r[1]], axis=1) * d1
        u2 = jnp.concatenate([u2r[0], u2r[1]], axis=1) * d2
        u3 = jnp.concatenate([u3r[0], u3r[1]], axis=1) * d1
        a = jnp.dot(amr[...], avr[...].T,
                    preferred_element_type=jnp.float32)
        w1 = jnp.dot(u1, a, preferred_element_type=jnp.float32)
        w2 = jnp.dot(u2, a, preferred_element_type=jnp.float32)
        w3 = jnp.dot(u3, a, preferred_element_type=jnp.float32)
        m = jnp.maximum(jnp.maximum(w1, w2), w3)
        e1 = jnp.exp(w1 - m)
        e2 = jnp.exp(w2 - m)
        e3 = jnp.exp(w3 - m)
        den = e1 + e2 + e3
        mixed = (u1 * e1 + u2 * e2 + u3 * e3) / den
        uf = jnp.concatenate([uir[0], uir[1]], axis=1) * du
        hm = _halves(mixed)
        o1[...] = _halves(u1)
        o2[...] = _halves(u2)
        o3[...] = _halves(u3)
        om[...] = hm
        oa[...] = uar[...] + hm + _halves(uf)

    return pl.pallas_call(
        body,
        grid=grid,
        in_specs=[tspec, tspec, tspec, tspec, dspec, mspec, vspec, tspec],
        out_specs=[tspec, tspec, tspec, tspec, tspec],
        out_shape=[oshape, oshape, oshape, oshape, oshape],
    )(u1a, u2a, u3a, uia, deg, am, av, uacc)


def _tc_item(ia, deg, iacc):
    grid = (PN // BLK,)
    tspec = pl.BlockSpec((NC, BLK, H), lambda r: (0, r, 0))
    dspec = pl.BlockSpec((4, BLK, 16), lambda r: (0, r, 0))
    oshape = jax.ShapeDtypeStruct((NC, PN, H), jnp.float32)

    def body(iar, dr, acr, oe, oa):
        di = 1.0 / jnp.maximum(dr[2, :, 0:1], 1.0)
        oe[...] = iar[...] * di[None]
        oa[...] = acr[...] + oe[...]

    return pl.pallas_call(
        body,
        grid=grid,
        in_specs=[tspec, dspec, tspec],
        out_specs=[tspec, tspec],
        out_shape=[oshape, oshape],
    )(ia, deg, iacc)


def _tc_final(uacc, iacc):
    grid = (PN // BLK,)
    tspec = pl.BlockSpec((NC, BLK, H), lambda r: (0, r, 0))
    ospec = pl.BlockSpec((BLK, D), lambda r: (r, 0))
    oshape = jax.ShapeDtypeStruct((PN, D), jnp.float32)

    def body(ur, ir, ou, oi):
        inv = 1.0 / NL
        ou[...] = jnp.concatenate([ur[0], ur[1]], axis=1) * inv
        oi[...] = jnp.concatenate([ir[0], ir[1]], axis=1) * inv

    return pl.pallas_call(
        body,
        grid=grid,
        in_specs=[tspec, tspec],
        out_specs=[ospec, ospec],
        out_shape=[oshape, oshape],
    )(uacc, iacc)


def kernel(user_emb, item_emb, gw1, gb1, gw2, gb2, gw3, gb3, sgw, sgb,
           att_mat, att_vec, social_src, social_dst, ui_user, ui_item):
    uep = jnp.zeros((PN, D), jnp.float32).at[:NU].set(user_emb)
    iep = jnp.zeros((PN, D), jnp.float32).at[:NI].set(item_emb)

    g_sf_s, g_sf_d, cs = _prep(social_src, social_dst)
    g_sb_s, g_sb_d, _ = _prep(social_dst, social_src)
    g_uf_s, g_uf_d, cu = _prep(ui_user, ui_item)
    g_ub_s, g_ub_d, _ = _prep(ui_item, ui_user)

    zeros32 = jnp.zeros((PN, H), jnp.float32)
    zeros16 = jnp.zeros((PN, 16), jnp.float32)
    ones16 = jnp.ones((128, 16), jnp.float32)

    u1, u2, u3, ie = _tc_gate(uep, iep, gw1, gb1, gw2, gb2, gw3, gb3,
                              sgw, sgb)
    deg = _sc_deg(cs, cu)(g_sf_d, g_sb_d, g_uf_d, g_ub_d, zeros16, ones16)

    prop_s = _sc_prop_multi([cs])
    prop_u = _sc_prop_multi([cu])

    uacc = jnp.zeros((NC, PN, H), jnp.float32)
    iacc = jnp.zeros((NC, PN, H), jnp.float32)
    for _ in range(NL):
        (u1a,) = prop_s(u1.reshape(NC * PN, H), g_sf_s, g_sf_d, zeros32)
        (u2a,) = prop_s(u2.reshape(NC * PN, H), g_sb_s, g_sb_d, zeros32)
        (u3a,) = prop_s(u3.reshape(NC * PN, H), g_sf_s, g_sf_d, zeros32)
        (uia,) = prop_u(ie.reshape(NC * PN, H), g_ub_s, g_ub_d, zeros32)
        u1, u2, u3, mixed, uacc = _tc_att(u1a, u2a, u3a, uia, deg,
                                          att_mat, att_vec, uacc)
        (ia,) = prop_u(mixed.reshape(NC * PN, H), g_uf_s, g_uf_d, zeros32)
        ie, iacc = _tc_item(ia, deg, iacc)

    up, ip = _tc_final(uacc, iacc)
    return jnp.concatenate([up[:NU], ip[:NI]], axis=0)
